# baseline (speedup 1.0000x reference)
import functools

import jax
import jax.numpy as jnp
from jax import lax
from jax.experimental import pallas as pl
from jax.experimental.pallas import tpu as pltpu
from jax.experimental.pallas import tpu_sc as plsc

N = 10000
E = 320000
B = 64
V = 100000
D = 128
NPAD = 10240
ZROW = 10200
NW = 32
EPW = 10112
EPAD = NW * EPW
NCHUNK = EPW // 128
TOTCH = EPAD // 128
NCH0 = 102
NCH1 = TOTCH // 16 - NCH0
RPS = NPAD // 16

_mesh = plsc.VectorSubcoreMesh(core_axis_name="c", subcore_axis_name="s")


@functools.partial(
    pl.kernel,
    out_type=jax.ShapeDtypeStruct((NPAD, D), jnp.float32),
    mesh=_mesh,
    scratch_types=[
        pltpu.VMEM((128,), jnp.int32),
        pltpu.VMEM((64,), jnp.int32),
        pltpu.VMEM((128, D), jnp.float32),
        pltpu.SemaphoreType.DMA,
    ],
)
def _sc_embed(tab_hbm, idx_hbm, out_hbm, idx_a, idx_b, rows_v, sem):
    del idx_b
    c = lax.axis_index("c")
    s = lax.axis_index("s")
    wid = s * 2 + c
    for t in range(3):
        cid = wid + NW * t

        @pl.when(cid < NPAD // 128)
        def _():
            base = cid * 128
            pltpu.sync_copy(idx_hbm.at[pl.ds(base, 128)], idx_a)
            pltpu.async_copy(tab_hbm.at[idx_a], rows_v, sem).wait()
            pltpu.sync_copy(rows_v, out_hbm.at[pl.ds(base, 128)])


@functools.partial(
    pl.kernel,
    out_type=[jax.ShapeDtypeStruct((NPAD, D), jnp.float32),
              jax.ShapeDtypeStruct((NPAD, D), jnp.float32)],
    mesh=_mesh,
    scratch_types=[
        pltpu.VMEM_SHARED((NPAD, D), jnp.float32),
        [pltpu.VMEM((128,), jnp.int32)] * 2,
        [pltpu.VMEM((128,), jnp.int32)] * 2,
        [pltpu.VMEM((128, D), jnp.float32)] * 2,
        [pltpu.SemaphoreType.DMA] * 2,
        [pltpu.SemaphoreType.DMA] * 2,
    ],
)
def _sc_rows(h_hbm, src_hbm, dst_hbm, zrow_hbm, out0, out1,
             accum, idx_s, idx_d, rows_v, sem, sem_s):
    c = lax.axis_index("c")
    s = lax.axis_index("s")
    wid = s * 2 + c

    pltpu.sync_copy(zrow_hbm, rows_v[0])
    for q in range(RPS // 128):
        pltpu.sync_copy(rows_v[0], accum.at[pl.ds(s * RPS + q * 128, 128)])
    plsc.subcore_barrier()

    cbase = jnp.where(c == 0, s, 16 * NCH0 + s)
    nch = jnp.where(c == 0, NCH0, NCH1)

    def stage(k, b):
        off = (cbase + k * 16) * 128
        pltpu.sync_copy(src_hbm.at[pl.ds(off, 128)], idx_s[b])
        pltpu.sync_copy(dst_hbm.at[pl.ds(off, 128)], idx_d[b])
        pltpu.async_copy(h_hbm.at[idx_s[b]], rows_v[b], sem[b])

    stage(0, 0)

    def body(k, carry):
        b = lax.rem(k, 2)
        bn = 1 - b

        @pl.when(k + 1 < nch)
        def _():
            for i in range(2):
                @pl.when(bn == i)
                def _():
                    @pl.when(k >= 1)
                    def _():
                        pltpu.make_async_copy(
                            rows_v[i], accum.at[idx_d[i]], sem_s[i]).wait()
                    stage(k + 1, i)

        for i in range(2):
            @pl.when(b == i)
            def _():
                pltpu.make_async_copy(h_hbm.at[idx_s[i]], rows_v[i],
                                      sem[i]).wait()
                pltpu.async_copy(rows_v[i], accum.at[idx_d[i]], sem_s[i],
                                 add=True)
        return carry

    lax.fori_loop(0, nch, body, 0)
    for i in range(2):
        pltpu.make_async_copy(rows_v[i], accum.at[idx_d[i]], sem_s[i]).wait()
    plsc.subcore_barrier()

    for q in range(RPS // 128):
        start = s * RPS + q * 128
        pltpu.sync_copy(accum.at[pl.ds(start, 128)], rows_v[0])

        @pl.when(c == 0)
        def _():
            pltpu.sync_copy(rows_v[0], out0.at[pl.ds(start, 128)])

        @pl.when(c == 1)
        def _():
            pltpu.sync_copy(rows_v[0], out1.at[pl.ds(start, 128)])


@functools.partial(
    pl.kernel,
    out_type=[jax.ShapeDtypeStruct((NPAD // 128, 128), jnp.float32),
              jax.ShapeDtypeStruct((NPAD // 128, 128), jnp.float32)],
    mesh=_mesh,
    scratch_types=[
        pltpu.VMEM_SHARED((16, NPAD // 128, 128), jnp.float32),
        pltpu.VMEM((128,), jnp.int32),
        pltpu.VMEM((128,), jnp.int32),
        pltpu.VMEM((NPAD // 128, 128), jnp.float32),
        pltpu.VMEM((NPAD // 128, 128), jnp.float32),
        pltpu.VMEM((16, 8, 128), jnp.float32),
        pltpu.SemaphoreType.DMA,
    ],
    compiler_params=pltpu.CompilerParams(needs_layout_passes=False),
)
def _sc_deg(nact_hbm, src_hbm, dst_hbm, zvec_hbm, outd0, outd1,
            degp, idx_s, idx_d, nact_t, deg_t, dsum_t, sem):
    del sem
    c = lax.axis_index("c")
    s = lax.axis_index("s")
    wid = s * 2 + c

    pltpu.sync_copy(nact_hbm, nact_t)
    pltpu.sync_copy(zvec_hbm, deg_t)

    base = wid * EPW

    def body(k, carry):
        off = base + k * 128
        pltpu.sync_copy(src_hbm.at[pl.ds(off, 128)], idx_s)
        pltpu.sync_copy(dst_hbm.at[pl.ds(off, 128)], idx_d)
        for q in range(8):
            si = idx_s[pl.ds(q * 16, 16)]
            di = idx_d[pl.ds(q * 16, 16)]
            nv = plsc.load_gather(nact_t, [si >> 7, si & 127])
            plsc.addupdate_scatter(deg_t, [di >> 7, di & 127], nv)
        return carry

    lax.fori_loop(0, NCHUNK, body, 0)
    pltpu.sync_copy(deg_t, degp.at[s])
    plsc.subcore_barrier()

    @pl.when(s < 10)
    def _():
        pltpu.sync_copy(degp.at[:, pl.ds(s * 8, 8)], dsum_t)
        for row in range(8):
            for q in range(8):
                acc = jnp.zeros((16,), jnp.float32)
                for r in range(16):
                    acc = acc + dsum_t[r, row, pl.ds(q * 16, 16)]
                deg_t[row, pl.ds(q * 16, 16)] = acc

        @pl.when(c == 0)
        def _():
            pltpu.sync_copy(deg_t.at[pl.ds(0, 8)], outd0.at[pl.ds(s * 8, 8)])

        @pl.when(c == 1)
        def _():
            pltpu.sync_copy(deg_t.at[pl.ds(0, 8)], outd1.at[pl.ds(s * 8, 8)])


_BR = 1024
_NBA = NPAD // _BR
_BK = 256
_NBK = NPAD // _BK


def _tc_dense_body(a0, a1, d0, d1, hp, na, bf, Wl, Wr, b, p,
                   hnew_ref, key_ref, score_ref):
    agg = a0[...] + a1[...]
    deg = d0[...] + d1[...]
    aggn = agg / jnp.maximum(deg, 1.0)
    h = hp[...]
    hnew = jnp.dot(aggn, Wl[...], precision=lax.Precision.HIGHEST)
    hnew = hnew + jnp.dot(h, Wr[...], precision=lax.Precision.HIGHEST)
    hnew = jnp.maximum(hnew + b[...], 0.0)
    pv = p[...]
    norm = jnp.sqrt(jnp.sum(pv * pv))
    score = jnp.dot(hnew, pv, precision=lax.Precision.HIGHEST) / (norm + 1e-16)
    key = bf[...] * 1e6 - jnp.where(na[...] > 0, score, -1e4)
    hnew_ref[...] = hnew
    key_ref[...] = key
    score_ref[...] = score


def _tc_dense(a0, a1, d0, d1, hp, na, bf, Wl, Wr, b, p):
    blk = lambda bs: pl.BlockSpec(bs, lambda i: (i, 0))
    full = lambda bs: pl.BlockSpec(bs, lambda i: (0, 0))
    return pl.pallas_call(
        _tc_dense_body,
        grid=(_NBA,),
        in_specs=[blk((_BR, D)), blk((_BR, D)), blk((_BR, 1)), blk((_BR, 1)),
                  blk((_BR, D)), blk((_BR, 1)), blk((_BR, 1)),
                  full((D, D)), full((D, D)), full((1, D)), full((D, 1))],
        out_specs=[blk((_BR, D)), blk((_BR, 1)), blk((_BR, 1))],
        out_shape=[jax.ShapeDtypeStruct((NPAD, D), jnp.float32),
                   jax.ShapeDtypeStruct((NPAD, 1), jnp.float32),
                   jax.ShapeDtypeStruct((NPAD, 1), jnp.float32)],
    )(a0, a1, d0, d1, hp, na, bf, Wl, Wr, b, p)


def _tc_keep_body(ki, bi, ii, ai, kj, bj, ij, aj, hnew, score,
                  keep_ref, h_ref, xcat_ref, macc, sacc, cacc):
    k_i = ki[...]
    b_i = bi[...]
    i_i = ii[...]
    a_i = ai[...]

    def jbody(j, carry):
        rank, cnt = carry
        k_j = kj[:, pl.ds(j * _BK, _BK)]
        b_j = bj[:, pl.ds(j * _BK, _BK)]
        i_j = ij[:, pl.ds(j * _BK, _BK)]
        a_j = aj[:, pl.ds(j * _BK, _BK)]
        sg = b_i == b_j
        less = (k_j < k_i) | ((k_j == k_i) & (i_j < i_i))
        rank = rank + jnp.sum(jnp.where(sg & less, 1.0, 0.0), axis=1,
                              keepdims=True)
        cnt = cnt + jnp.sum(jnp.where(sg & (a_j > 0), 1.0, 0.0), axis=1,
                            keepdims=True)
        return rank, cnt

    bmin = jnp.min(b_i)
    bmax = jnp.max(b_i)
    bj_all = bj[...]
    lo = jnp.sum(jnp.where(bj_all < bmin, 1.0, 0.0)).astype(jnp.int32)
    hi = jnp.sum(jnp.where(bj_all <= bmax, 1.0, 0.0)).astype(jnp.int32)
    jlo = lax.shift_right_logical(lo, 8)
    jhi = lax.shift_right_logical(hi + _BK - 1, 8)

    zero = jnp.zeros((_BK, 1), jnp.float32)
    rank, cnt = lax.fori_loop(jlo, jhi, jbody, (zero, zero))
    kv = jnp.ceil(jnp.float32(0.8) * cnt)
    kp = jnp.where((a_i > 0) & (rank < kv), 1.0, 0.0)
    keep_ref[...] = kp

    i = pl.program_id(0)

    @pl.when(i == 0)
    def _():
        macc[...] = jnp.full((B, D), -1e30, jnp.float32)
        sacc[...] = jnp.zeros((B, D), jnp.float32)
        cacc[...] = jnp.zeros((B, 1), jnp.float32)

    h = hnew[...] * jnp.tanh(score[...]) * kp
    h_ref[...] = h

    gids = lax.broadcasted_iota(jnp.int32, (B, _BK), 0).astype(jnp.float32)
    oh = jnp.where(gids == bj[:, pl.ds(i * _BK, _BK)], 1.0, 0.0)
    sacc[...] = sacc[...] + jnp.dot(oh, h, precision=lax.Precision.HIGHEST)
    cacc[...] = cacc[...] + jnp.dot(oh, kp, precision=lax.Precision.HIGHEST)

    def gbody(g, carry):
        mask = (b_i == g.astype(jnp.float32)) & (kp > 0)
        mg = jnp.max(jnp.where(mask, h, -1e30), axis=0, keepdims=True)
        macc[pl.ds(g, 1), :] = jnp.maximum(macc[pl.ds(g, 1), :], mg)
        return carry

    gmin = bmin.astype(jnp.int32)
    gmax = bmax.astype(jnp.int32)
    lax.fori_loop(gmin, gmax + 1, gbody, 0)

    @pl.when(i == _NBK - 1)
    def _():
        m = macc[...]
        gmp = jnp.where(m > -1e29, m, 0.0)
        gap = sacc[...] / jnp.maximum(cacc[...], 1.0)
        xcat_ref[:, :D] = gmp
        xcat_ref[:, D:] = gap


def _tc_keep(key, batf, idxf, act, hnew, score):
    kj = key.reshape(1, NPAD)
    bj = batf.reshape(1, NPAD)
    ij = idxf.reshape(1, NPAD)
    aj = act.reshape(1, NPAD)
    blk = pl.BlockSpec((_BK, 1), lambda i: (i, 0))
    row = pl.BlockSpec((1, NPAD), lambda i: (0, 0))
    blkd = pl.BlockSpec((_BK, D), lambda i: (i, 0))
    return pl.pallas_call(
        _tc_keep_body,
        grid=(_NBK,),
        in_specs=[blk, blk, blk, blk, row, row, row, row, blkd, blk],
        out_specs=[blk, blkd, pl.BlockSpec((B, 2 * D), lambda i: (0, 0))],
        out_shape=[jax.ShapeDtypeStruct((NPAD, 1), jnp.float32),
                   jax.ShapeDtypeStruct((NPAD, D), jnp.float32),
                   jax.ShapeDtypeStruct((B, 2 * D), jnp.float32)],
        scratch_shapes=[pltpu.VMEM((B, D), jnp.float32),
                        pltpu.VMEM((B, D), jnp.float32),
                        pltpu.VMEM((B, 1), jnp.float32)],
    )(key, batf, idxf, act, kj, bj, ij, aj, hnew, score)


def _tc_mlp_body(x1, x2, x3, W1, b1, W2, b2, W3, b3, out_ref):
    z = x1[...] + x2[...] + x3[...]
    z = jnp.maximum(jnp.dot(z, W1[...], precision=lax.Precision.HIGHEST)
                    + b1[...], 0.0)
    z = jnp.maximum(jnp.dot(z, W2[...], precision=lax.Precision.HIGHEST)
                    + b2[...], 0.0)
    z = jnp.dot(z, W3[...], precision=lax.Precision.HIGHEST) + b3[...]
    out_ref[...] = jax.nn.sigmoid(z)


def _tc_mlp(x1, x2, x3, W1, b1, W2, b2, W3, b3):
    return pl.pallas_call(
        _tc_mlp_body,
        out_shape=jax.ShapeDtypeStruct((B, 1), jnp.float32),
    )(x1, x2, x3, W1, b1, W2, b2, W3, b3)


def kernel(x, edge_index, batch, emb, Wl1, Wr1, bc1, p1, Wl2, Wr2, bc2, p2,
           Wl3, Wr3, bc3, p3, W1, bl1, W2, bl2, W3, bl3):
    f32 = jnp.float32

    idx0 = jnp.concatenate(
        [x[:, 0].astype(jnp.int32), jnp.full((NPAD - N,), V, jnp.int32)])
    src = jnp.concatenate(
        [edge_index[0].astype(jnp.int32),
         jnp.full((EPAD - E,), ZROW, jnp.int32)])
    dst = jnp.concatenate(
        [edge_index[1].astype(jnp.int32),
         jnp.full((EPAD - E,), ZROW, jnp.int32)])
    batf = jnp.concatenate(
        [batch.astype(f32), jnp.full((NPAD - N,), float(B - 1), f32)]
    ).reshape(NPAD, 1)
    idxf = jnp.arange(NPAD, dtype=f32).reshape(NPAD, 1)
    embA = jnp.concatenate([emb.astype(f32), jnp.zeros((16, D), f32)], axis=0)
    zrow = jnp.zeros((128, D), f32)
    zvec = jnp.zeros((NPAD // 128, 128), f32)
    nact = jnp.concatenate([jnp.ones((N,), f32), jnp.zeros((NPAD - N,), f32)])

    h = _sc_embed(embA, idx0)

    params = [(Wl1, Wr1, bc1, p1), (Wl2, Wr2, bc2, p2), (Wl3, Wr3, bc3, p3)]
    xs = []
    for Wl, Wr, bc, p in params:
        acc0, acc1 = _sc_rows(h, src, dst, zrow)
        dg0, dg1 = _sc_deg(nact.reshape(NPAD // 128, 128), src, dst, zvec)
        na2 = nact.reshape(NPAD, 1)
        hnew, key, score = _tc_dense(
            acc0, acc1, dg0.reshape(NPAD, 1), dg1.reshape(NPAD, 1),
            h, na2, batf, Wl, Wr, bc.reshape(1, D), p.reshape(D, 1))
        keep, h, xcat = _tc_keep(key, batf, idxf, na2, hnew, score)
        nact = keep[:, 0]
        xs.append(xcat)

    out = _tc_mlp(xs[0], xs[1], xs[2],
                  W1, bl1.reshape(1, D), W2, bl2.reshape(1, 64),
                  W3, bl3.reshape(1, 1))
    return out[:, 0]


# speedup vs baseline: 1.0333x; 1.0333x over previous
import functools

import jax
import jax.numpy as jnp
from jax import lax
from jax.experimental import pallas as pl
from jax.experimental.pallas import tpu as pltpu
from jax.experimental.pallas import tpu_sc as plsc

N = 10000
E = 320000
B = 64
V = 100000
D = 128
NPAD = 10240
ZROW = 10200
NW = 32
EPW = 10112
EPAD = NW * EPW
NCHUNK = EPW // 128
TOTCH = EPAD // 128
NCH0 = 102
NCH1 = TOTCH // 16 - NCH0
RPS = NPAD // 16

_mesh = plsc.VectorSubcoreMesh(core_axis_name="c", subcore_axis_name="s")


@functools.partial(
    pl.kernel,
    out_type=jax.ShapeDtypeStruct((NPAD, D), jnp.float32),
    mesh=_mesh,
    scratch_types=[
        pltpu.VMEM((128,), jnp.int32),
        pltpu.VMEM((64,), jnp.int32),
        pltpu.VMEM((128, D), jnp.float32),
        pltpu.SemaphoreType.DMA,
    ],
)
def _sc_embed(tab_hbm, idx_hbm, out_hbm, idx_a, idx_b, rows_v, sem):
    del idx_b
    c = lax.axis_index("c")
    s = lax.axis_index("s")
    wid = s * 2 + c
    for t in range(3):
        cid = wid + NW * t

        @pl.when(cid < NPAD // 128)
        def _():
            base = cid * 128
            pltpu.sync_copy(idx_hbm.at[pl.ds(base, 128)], idx_a)
            pltpu.async_copy(tab_hbm.at[idx_a], rows_v, sem).wait()
            pltpu.sync_copy(rows_v, out_hbm.at[pl.ds(base, 128)])


@functools.partial(
    pl.kernel,
    out_type=[jax.ShapeDtypeStruct((NPAD, D), jnp.float32),
              jax.ShapeDtypeStruct((NPAD, D), jnp.float32)],
    mesh=_mesh,
    scratch_types=[
        pltpu.VMEM_SHARED((NPAD, D), jnp.float32),
        [pltpu.VMEM((128,), jnp.int32)] * 2,
        [pltpu.VMEM((128,), jnp.int32)] * 2,
        [pltpu.VMEM((128, D), jnp.float32)] * 2,
        [pltpu.SemaphoreType.DMA] * 2,
        [pltpu.SemaphoreType.DMA] * 2,
    ],
)
def _sc_rows(h_hbm, src_hbm, dst_hbm, zrow_hbm, out0, out1,
             accum, idx_s, idx_d, rows_v, sem, sem_s):
    c = lax.axis_index("c")
    s = lax.axis_index("s")
    wid = s * 2 + c

    pltpu.sync_copy(zrow_hbm, rows_v[0])
    for q in range(RPS // 128):
        pltpu.sync_copy(rows_v[0], accum.at[pl.ds(s * RPS + q * 128, 128)])
    plsc.subcore_barrier()

    cbase = jnp.where(c == 0, s, 16 * NCH0 + s)
    nch = jnp.where(c == 0, NCH0, NCH1)

    def stage(k, b):
        off = (cbase + k * 16) * 128
        pltpu.sync_copy(src_hbm.at[pl.ds(off, 128)], idx_s[b])
        pltpu.sync_copy(dst_hbm.at[pl.ds(off, 128)], idx_d[b])
        pltpu.async_copy(h_hbm.at[idx_s[b]], rows_v[b], sem[b])

    stage(0, 0)

    def body(k, carry):
        b = lax.rem(k, 2)
        bn = 1 - b

        @pl.when(k + 1 < nch)
        def _():
            for i in range(2):
                @pl.when(bn == i)
                def _():
                    @pl.when(k >= 1)
                    def _():
                        pltpu.make_async_copy(
                            rows_v[i], accum.at[idx_d[i]], sem_s[i]).wait()
                    stage(k + 1, i)

        for i in range(2):
            @pl.when(b == i)
            def _():
                pltpu.make_async_copy(h_hbm.at[idx_s[i]], rows_v[i],
                                      sem[i]).wait()
                pltpu.async_copy(rows_v[i], accum.at[idx_d[i]], sem_s[i],
                                 add=True)
        return carry

    lax.fori_loop(0, nch, body, 0)
    for i in range(2):
        pltpu.make_async_copy(rows_v[i], accum.at[idx_d[i]], sem_s[i]).wait()
    plsc.subcore_barrier()

    for q in range(RPS // 128):
        start = s * RPS + q * 128
        pltpu.sync_copy(accum.at[pl.ds(start, 128)], rows_v[0])

        @pl.when(c == 0)
        def _():
            pltpu.sync_copy(rows_v[0], out0.at[pl.ds(start, 128)])

        @pl.when(c == 1)
        def _():
            pltpu.sync_copy(rows_v[0], out1.at[pl.ds(start, 128)])


@functools.partial(
    pl.kernel,
    out_type=[jax.ShapeDtypeStruct((NPAD // 128, 128), jnp.float32),
              jax.ShapeDtypeStruct((NPAD // 128, 128), jnp.float32)],
    mesh=_mesh,
    scratch_types=[
        pltpu.VMEM_SHARED((16, NPAD // 128, 128), jnp.float32),
        pltpu.VMEM((128,), jnp.int32),
        pltpu.VMEM((128,), jnp.int32),
        pltpu.VMEM((NPAD // 128, 128), jnp.float32),
        pltpu.VMEM((NPAD // 128, 128), jnp.float32),
        pltpu.VMEM((16, 8, 128), jnp.float32),
        pltpu.SemaphoreType.DMA,
    ],
    compiler_params=pltpu.CompilerParams(needs_layout_passes=False),
)
def _sc_deg(nact_hbm, src_hbm, dst_hbm, zvec_hbm, outd0, outd1,
            degp, idx_s, idx_d, nact_t, deg_t, dsum_t, sem):
    del sem
    c = lax.axis_index("c")
    s = lax.axis_index("s")
    wid = s * 2 + c

    pltpu.sync_copy(nact_hbm, nact_t)
    pltpu.sync_copy(zvec_hbm, deg_t)

    base = wid * EPW

    def body(k, carry):
        off = base + k * 128
        pltpu.sync_copy(src_hbm.at[pl.ds(off, 128)], idx_s)
        pltpu.sync_copy(dst_hbm.at[pl.ds(off, 128)], idx_d)
        for q in range(8):
            si = idx_s[pl.ds(q * 16, 16)]
            di = idx_d[pl.ds(q * 16, 16)]
            nv = plsc.load_gather(nact_t, [si >> 7, si & 127])
            plsc.addupdate_scatter(deg_t, [di >> 7, di & 127], nv)
        return carry

    lax.fori_loop(0, NCHUNK, body, 0)
    pltpu.sync_copy(deg_t, degp.at[s])
    plsc.subcore_barrier()

    @pl.when(s < 10)
    def _():
        pltpu.sync_copy(degp.at[:, pl.ds(s * 8, 8)], dsum_t)
        for row in range(8):
            for q in range(8):
                acc = jnp.zeros((16,), jnp.float32)
                for r in range(16):
                    acc = acc + dsum_t[r, row, pl.ds(q * 16, 16)]
                deg_t[row, pl.ds(q * 16, 16)] = acc

        @pl.when(c == 0)
        def _():
            pltpu.sync_copy(deg_t.at[pl.ds(0, 8)], outd0.at[pl.ds(s * 8, 8)])

        @pl.when(c == 1)
        def _():
            pltpu.sync_copy(deg_t.at[pl.ds(0, 8)], outd1.at[pl.ds(s * 8, 8)])


_BR = 1024
_NBA = NPAD // _BR
_BK = 256
_NBK = NPAD // _BK


def _tc_dense_body(a0, a1, d0, d1, hp, na, bf, Wl, Wr, b, p,
                   hnew_ref, key_ref, score_ref):
    agg = a0[...] + a1[...]
    deg = d0[...] + d1[...]
    aggn = agg / jnp.maximum(deg, 1.0)
    h = hp[...]
    hnew = jnp.dot(aggn, Wl[...], precision=lax.Precision.HIGHEST)
    hnew = hnew + jnp.dot(h, Wr[...], precision=lax.Precision.HIGHEST)
    hnew = jnp.maximum(hnew + b[...], 0.0)
    pv = p[...]
    norm = jnp.sqrt(jnp.sum(pv * pv))
    score = jnp.dot(hnew, pv, precision=lax.Precision.HIGHEST) / (norm + 1e-16)
    key = bf[...] * 1e6 - jnp.where(na[...] > 0, score, -1e4)
    hnew_ref[...] = hnew
    key_ref[...] = key
    score_ref[...] = score


def _tc_dense(a0, a1, d0, d1, hp, na, bf, Wl, Wr, b, p):
    blk = lambda bs: pl.BlockSpec(bs, lambda i: (i, 0))
    full = lambda bs: pl.BlockSpec(bs, lambda i: (0, 0))
    return pl.pallas_call(
        _tc_dense_body,
        grid=(_NBA,),
        in_specs=[blk((_BR, D)), blk((_BR, D)), blk((_BR, 1)), blk((_BR, 1)),
                  blk((_BR, D)), blk((_BR, 1)), blk((_BR, 1)),
                  full((D, D)), full((D, D)), full((1, D)), full((D, 1))],
        out_specs=[blk((_BR, D)), blk((_BR, 1)), blk((_BR, 1))],
        out_shape=[jax.ShapeDtypeStruct((NPAD, D), jnp.float32),
                   jax.ShapeDtypeStruct((NPAD, 1), jnp.float32),
                   jax.ShapeDtypeStruct((NPAD, 1), jnp.float32)],
    )(a0, a1, d0, d1, hp, na, bf, Wl, Wr, b, p)


def _tc_keep_body(ki, bi, ii, ai, kj, bj, ij, aj, keep_ref):
    k_i = ki[...]
    b_i = bi[...]
    i_i = ii[...]
    a_i = ai[...]

    def jbody(j, carry):
        rank, cnt = carry
        k_j = kj[:, pl.ds(j * _BK, _BK)]
        b_j = bj[:, pl.ds(j * _BK, _BK)]
        i_j = ij[:, pl.ds(j * _BK, _BK)]
        a_j = aj[:, pl.ds(j * _BK, _BK)]
        sg = b_i == b_j
        less = (k_j < k_i) | ((k_j == k_i) & (i_j < i_i))
        rank = rank + jnp.sum(jnp.where(sg & less, 1.0, 0.0), axis=1,
                              keepdims=True)
        cnt = cnt + jnp.sum(jnp.where(sg & (a_j > 0), 1.0, 0.0), axis=1,
                            keepdims=True)
        return rank, cnt

    bmin = jnp.min(b_i)
    bmax = jnp.max(b_i)
    bj_all = bj[...]
    lo = jnp.sum(jnp.where(bj_all < bmin, 1.0, 0.0)).astype(jnp.int32)
    hi = jnp.sum(jnp.where(bj_all <= bmax, 1.0, 0.0)).astype(jnp.int32)
    jlo = lax.shift_right_logical(lo, 8)
    jhi = lax.shift_right_logical(hi + _BK - 1, 8)

    zero = jnp.zeros((_BK, 1), jnp.float32)
    rank, cnt = lax.fori_loop(jlo, jhi, jbody, (zero, zero))
    kv = jnp.ceil(jnp.float32(0.8) * cnt)
    keep_ref[...] = jnp.where((a_i > 0) & (rank < kv), 1.0, 0.0)


def _tc_keep(key, batf, idxf, act):
    kj = key.reshape(1, NPAD)
    bj = batf.reshape(1, NPAD)
    ij = idxf.reshape(1, NPAD)
    aj = act.reshape(1, NPAD)
    blk = pl.BlockSpec((_BK, 1), lambda i: (i, 0))
    row = pl.BlockSpec((1, NPAD), lambda i: (0, 0))
    return pl.pallas_call(
        _tc_keep_body,
        grid=(_NBK,),
        in_specs=[blk, blk, blk, blk, row, row, row, row],
        out_specs=blk,
        out_shape=jax.ShapeDtypeStruct((NPAD, 1), jnp.float32),
    )(key, batf, idxf, act, kj, bj, ij, aj)


def _tc_apply_body(hnew, score, keep, bi, bj, h_ref, xcat_ref,
                   macc, sacc, cacc):
    i = pl.program_id(0)

    @pl.when(i == 0)
    def _():
        macc[...] = jnp.full((B, D), -1e30, jnp.float32)
        sacc[...] = jnp.zeros((B, D), jnp.float32)
        cacc[...] = jnp.zeros((B, 1), jnp.float32)

    kp = keep[...]
    h = hnew[...] * jnp.tanh(score[...]) * kp
    h_ref[...] = h

    gids = lax.broadcasted_iota(jnp.int32, (B, _BR), 0).astype(jnp.float32)
    oh = jnp.where(gids == bj[...], 1.0, 0.0)
    sacc[...] = sacc[...] + jnp.dot(oh, h, precision=lax.Precision.HIGHEST)
    cacc[...] = cacc[...] + jnp.dot(oh, kp, precision=lax.Precision.HIGHEST)

    b_i = bi[...]

    def gbody(g, carry):
        mask = (b_i == g.astype(jnp.float32)) & (kp > 0)
        mg = jnp.max(jnp.where(mask, h, -1e30), axis=0, keepdims=True)
        macc[pl.ds(g, 1), :] = jnp.maximum(macc[pl.ds(g, 1), :], mg)
        return carry

    gmin = jnp.min(b_i).astype(jnp.int32)
    gmax = jnp.max(b_i).astype(jnp.int32)
    lax.fori_loop(gmin, gmax + 1, gbody, 0)

    @pl.when(i == _NBA - 1)
    def _():
        m = macc[...]
        gmp = jnp.where(m > -1e29, m, 0.0)
        gap = sacc[...] / jnp.maximum(cacc[...], 1.0)
        xcat_ref[:, :D] = gmp
        xcat_ref[:, D:] = gap


def _tc_apply(hnew, score, keep, batf):
    bj = batf.reshape(1, NPAD)
    blk = lambda bs: pl.BlockSpec(bs, lambda i: (i, 0))
    return pl.pallas_call(
        _tc_apply_body,
        grid=(_NBA,),
        in_specs=[blk((_BR, D)), blk((_BR, 1)), blk((_BR, 1)), blk((_BR, 1)),
                  pl.BlockSpec((1, _BR), lambda i: (0, i))],
        out_specs=[blk((_BR, D)), pl.BlockSpec((B, 2 * D), lambda i: (0, 0))],
        out_shape=[jax.ShapeDtypeStruct((NPAD, D), jnp.float32),
                   jax.ShapeDtypeStruct((B, 2 * D), jnp.float32)],
        scratch_shapes=[pltpu.VMEM((B, D), jnp.float32),
                        pltpu.VMEM((B, D), jnp.float32),
                        pltpu.VMEM((B, 1), jnp.float32)],
    )(hnew, score, keep, batf, bj)


def _tc_mlp_body(x1, x2, x3, W1, b1, W2, b2, W3, b3, out_ref):
    z = x1[...] + x2[...] + x3[...]
    z = jnp.maximum(jnp.dot(z, W1[...], precision=lax.Precision.HIGHEST)
                    + b1[...], 0.0)
    z = jnp.maximum(jnp.dot(z, W2[...], precision=lax.Precision.HIGHEST)
                    + b2[...], 0.0)
    z = jnp.dot(z, W3[...], precision=lax.Precision.HIGHEST) + b3[...]
    out_ref[...] = jax.nn.sigmoid(z)


def _tc_mlp(x1, x2, x3, W1, b1, W2, b2, W3, b3):
    return pl.pallas_call(
        _tc_mlp_body,
        out_shape=jax.ShapeDtypeStruct((B, 1), jnp.float32),
    )(x1, x2, x3, W1, b1, W2, b2, W3, b3)


def kernel(x, edge_index, batch, emb, Wl1, Wr1, bc1, p1, Wl2, Wr2, bc2, p2,
           Wl3, Wr3, bc3, p3, W1, bl1, W2, bl2, W3, bl3):
    f32 = jnp.float32

    idx0 = jnp.concatenate(
        [x[:, 0].astype(jnp.int32), jnp.full((NPAD - N,), V, jnp.int32)])
    src = jnp.concatenate(
        [edge_index[0].astype(jnp.int32),
         jnp.full((EPAD - E,), ZROW, jnp.int32)])
    dst = jnp.concatenate(
        [edge_index[1].astype(jnp.int32),
         jnp.full((EPAD - E,), ZROW, jnp.int32)])
    batf = jnp.concatenate(
        [batch.astype(f32), jnp.full((NPAD - N,), float(B - 1), f32)]
    ).reshape(NPAD, 1)
    idxf = jnp.arange(NPAD, dtype=f32).reshape(NPAD, 1)
    embA = jnp.concatenate([emb.astype(f32), jnp.zeros((16, D), f32)], axis=0)
    zrow = jnp.zeros((128, D), f32)
    zvec = jnp.zeros((NPAD // 128, 128), f32)
    nact = jnp.concatenate([jnp.ones((N,), f32), jnp.zeros((NPAD - N,), f32)])

    h = _sc_embed(embA, idx0)

    params = [(Wl1, Wr1, bc1, p1), (Wl2, Wr2, bc2, p2), (Wl3, Wr3, bc3, p3)]
    xs = []
    for Wl, Wr, bc, p in params:
        acc0, acc1 = _sc_rows(h, src, dst, zrow)
        dg0, dg1 = _sc_deg(nact.reshape(NPAD // 128, 128), src, dst, zvec)
        na2 = nact.reshape(NPAD, 1)
        hnew, key, score = _tc_dense(
            acc0, acc1, dg0.reshape(NPAD, 1), dg1.reshape(NPAD, 1),
            h, na2, batf, Wl, Wr, bc.reshape(1, D), p.reshape(D, 1))
        keep = _tc_keep(key, batf, idxf, na2)
        h, xcat = _tc_apply(hnew, score, keep, batf)
        nact = keep[:, 0]
        xs.append(xcat)

    out = _tc_mlp(xs[0], xs[1], xs[2],
                  W1, bl1.reshape(1, D), W2, bl2.reshape(1, 64),
                  W3, bl3.reshape(1, 1))
    return out[:, 0]
